# baseline (speedup 1.0000x reference)
import functools

import jax
import jax.numpy as jnp
from jax import lax
from jax.experimental import pallas as pl
from jax.experimental.pallas import tpu as pltpu
from jax.experimental.pallas import tpu_sc as plsc

VOCAB = 100000
EMBED_DIM = 128
LANES = 16
CHUNK = 128


def _make_kernel(batch, seq_len):
    info = plsc.get_sparse_core_info()
    nc, ns = info.num_cores, info.num_subcores
    nw = nc * ns
    total = batch * seq_len
    per_w = total // nw
    n_chunks = per_w // CHUNK
    w_per_row = seq_len // per_w

    mesh = plsc.VectorSubcoreMesh(core_axis_name="c", subcore_axis_name="s")

    @functools.partial(
        pl.kernel,
        mesh=mesh,
        out_type=jax.ShapeDtypeStruct((total, EMBED_DIM), jnp.float32),
        scratch_types=[
            pltpu.VMEM((1, per_w), jnp.int32),
            pltpu.VMEM((per_w, EMBED_DIM), jnp.float32),
            pltpu.VMEM((n_chunks, CHUNK), jnp.int32),
            pltpu.VMEM_SHARED((ns * per_w, EMBED_DIM), jnp.float32),
        ]
        + [pltpu.SemaphoreType.DMA] * (2 * n_chunks)
        + [pltpu.SemaphoreType.DMA, pltpu.SemaphoreType.DMA,
           pltpu.SemaphoreType.DMA],
    )
    def emb(x_hbm, tok_hbm, pos_hbm, out_hbm,
            idx_v, rows_v, sidx_v, shared, *sems):
        g_sems = sems[:n_chunks]
        a_sems = sems[n_chunks:2 * n_chunks]
        pos_sem, out_sem, idx_sem = sems[2 * n_chunks:]
        sid = lax.axis_index("s")
        wid = sid * nc + lax.axis_index("c")
        base = wid * per_w
        b = lax.div(wid, w_per_row)
        s0 = lax.rem(base, seq_len)
        blk = sid * per_w

        pltpu.sync_copy(x_hbm.at[pl.ds(b, 1), pl.ds(s0, per_w)], idx_v)
        gathers = []
        for i in range(n_chunks):
            sl = pl.ds(i * CHUNK, CHUNK)
            gathers.append(
                pltpu.async_copy(
                    tok_hbm.at[idx_v.at[0, sl]], rows_v.at[sl], g_sems[i]))
        pos_fill = pltpu.async_copy(
            pos_hbm.at[pl.ds(s0, per_w)],
            shared.at[pl.ds(blk, per_w)], pos_sem)
        for i in range(n_chunks):
            for j in range(CHUNK // LANES):
                sidx_v[i, pl.ds(j * LANES, LANES)] = (
                    lax.iota(jnp.int32, LANES) + (blk + i * CHUNK + j * LANES))
        pos_fill.wait()
        adds = []
        for i in range(n_chunks):
            sl = pl.ds(i * CHUNK, CHUNK)
            gathers[i].wait()
            adds.append(
                pltpu.async_copy(
                    rows_v.at[sl], shared.at[sidx_v.at[i]], a_sems[i],
                    add=True))
        stores = []
        for i in range(n_chunks):
            adds[i].wait()
            stores.append(
                pltpu.async_copy(
                    shared.at[pl.ds(blk + i * CHUNK, CHUNK)],
                    out_hbm.at[pl.ds(base + i * CHUNK, CHUNK)], out_sem))
        for st in stores:
            st.wait()

    return emb


@jax.jit
def kernel(x, token_table, position_table):
    batch, seq_len = x.shape
    emb = _make_kernel(batch, seq_len)
    out = emb(x, token_table, position_table)
    return out.reshape(batch, seq_len, EMBED_DIM)


# speedup vs baseline: 1.0380x; 1.0380x over previous
import functools

import jax
import jax.numpy as jnp
from jax import lax
from jax.experimental import pallas as pl
from jax.experimental.pallas import tpu as pltpu
from jax.experimental.pallas import tpu_sc as plsc

VOCAB = 100000
EMBED_DIM = 128
LANES = 16
CHUNK = 128


def _make_kernel(batch, seq_len):
    info = plsc.get_sparse_core_info()
    nc, ns = info.num_cores, info.num_subcores
    nw = nc * ns
    total = batch * seq_len
    per_w = total // nw
    n_chunks = per_w // CHUNK
    w_per_row = seq_len // per_w

    mesh = plsc.VectorSubcoreMesh(core_axis_name="c", subcore_axis_name="s")

    @functools.partial(
        pl.kernel,
        mesh=mesh,
        out_type=jax.ShapeDtypeStruct((total, EMBED_DIM), jnp.float32),
        scratch_types=[
            pltpu.VMEM((1, per_w), jnp.int32),
            pltpu.VMEM((per_w, EMBED_DIM), jnp.float32),
            pltpu.VMEM((n_chunks, CHUNK), jnp.int32),
            pltpu.VMEM_SHARED((ns * per_w, EMBED_DIM), jnp.float32),
        ]
        + [pltpu.SemaphoreType.DMA] * n_chunks
        + [pltpu.SemaphoreType.DMA, pltpu.SemaphoreType.DMA],
    )
    def emb(x_hbm, tok_hbm, pos_hbm, out_hbm,
            idx_v, rows_v, sidx_v, shared, *sems):
        g_sems = sems[:n_chunks]
        pos_sem, out_sem = sems[n_chunks], sems[n_chunks + 1]
        sid = lax.axis_index("s")
        wid = sid * nc + lax.axis_index("c")
        base = wid * per_w
        b = lax.div(wid, w_per_row)
        s0 = lax.rem(base, seq_len)
        blk = sid * per_w

        pltpu.sync_copy(x_hbm.at[pl.ds(b, 1), pl.ds(s0, per_w)], idx_v)
        gathers = []
        for i in range(n_chunks):
            sl = pl.ds(i * CHUNK, CHUNK)
            gathers.append(
                pltpu.async_copy(
                    tok_hbm.at[idx_v.at[0, sl]], rows_v.at[sl], g_sems[i]))
        pos_fill = pltpu.async_copy(
            pos_hbm.at[pl.ds(s0, per_w)],
            shared.at[pl.ds(blk, per_w)], pos_sem)
        for i in range(n_chunks):
            for j in range(CHUNK // LANES):
                sidx_v[i, pl.ds(j * LANES, LANES)] = (
                    lax.iota(jnp.int32, LANES) + (blk + i * CHUNK + j * LANES))
        pos_fill.wait()
        stores = []
        for i in range(n_chunks):
            sl = pl.ds(i * CHUNK, CHUNK)
            gathers[i].wait()
            pltpu.sync_copy(rows_v.at[sl], shared.at[sidx_v.at[i]], add=True)
            stores.append(
                pltpu.async_copy(
                    shared.at[pl.ds(blk + i * CHUNK, CHUNK)],
                    out_hbm.at[pl.ds(base + i * CHUNK, CHUNK)], out_sem))
        for st in stores:
            st.wait()

    return emb


@jax.jit
def kernel(x, token_table, position_table):
    batch, seq_len = x.shape
    emb = _make_kernel(batch, seq_len)
    out = emb(x, token_table, position_table)
    return out.reshape(batch, seq_len, EMBED_DIM)
